# baseline (speedup 1.0000x reference)
import functools

import jax
import jax.numpy as jnp
from jax import lax
from jax.experimental import pallas as pl
from jax.experimental.pallas import tpu as pltpu
from jax.experimental.pallas import tpu_sc as plsc

MAXD = 10
EDIM = 64
BATCH = 4096
EPS = 1e-5

NC, NS = 2, 16
NW = NC * NS
NIDX = BATCH * MAXD
CHUNK = 128
NCHUNK = NIDX // (NW * CHUNK)


def _gather_body(idx_hbm, table_hbm, out_hbm, idx_v, rows_v, sem):
    wid = lax.axis_index("s") * NC + lax.axis_index("c")
    pltpu.sync_copy(idx_hbm.at[wid], idx_v)
    copies = [
        pltpu.async_copy(table_hbm.at[idx_v.at[j]], rows_v.at[j], sem)
        for j in range(NCHUNK)
    ]
    for c in copies:
        c.wait()
    pltpu.sync_copy(rows_v, out_hbm.at[wid])


@functools.cache
def _build_sc_gather():
    return pl.kernel(
        _gather_body,
        out_type=jax.ShapeDtypeStruct((NW, NCHUNK, CHUNK, EDIM), jnp.float32),
        mesh=plsc.VectorSubcoreMesh(
            core_axis_name="c", subcore_axis_name="s",
            num_cores=NC, num_subcores=NS,
        ),
        scratch_types=[
            pltpu.VMEM((NCHUNK, CHUNK), jnp.int32),
            pltpu.VMEM((NCHUNK, CHUNK, EDIM), jnp.float32),
            pltpu.SemaphoreType.DMA,
        ],
        compiler_params=pltpu.CompilerParams(use_tc_tiling_on_sc=False),
    )


def _bn_relu(h, g, be):
    m = jnp.mean(h, axis=0, keepdims=True)
    c = h - m
    v = jnp.mean(c * c, axis=0, keepdims=True)
    return jnp.maximum(g * c * jax.lax.rsqrt(v + EPS) + be, 0.0)


def _mlp_body(e_ref, num_ref, w1e_ref, w1n_ref, b1_ref, g1_ref, be1_ref,
              w2_ref, b2_ref, g2_ref, be2_ref,
              w3_ref, b3_ref, g3_ref, be3_ref,
              wo_ref, bo_ref, out_ref):
    f32 = jnp.float32
    h1 = (jnp.dot(e_ref[...], w1e_ref[...], preferred_element_type=f32)
          + jnp.dot(num_ref[...], w1n_ref[...], preferred_element_type=f32)
          + b1_ref[...])
    h1 = _bn_relu(h1, g1_ref[...], be1_ref[...])
    h2 = jnp.dot(h1, w2_ref[...], preferred_element_type=f32) + b2_ref[...]
    h2 = _bn_relu(h2, g2_ref[...], be2_ref[...])
    h3 = jnp.dot(h2, w3_ref[...], preferred_element_type=f32) + b3_ref[...]
    h3 = _bn_relu(h3, g3_ref[...], be3_ref[...])
    out_ref[...] = (jnp.dot(h3, wo_ref[...], preferred_element_type=f32)
                    + bo_ref[...])


def kernel(x, emb, W1, b1, g1, be1, W2, b2, g2, be2, W3, b3, g3, be3, Wo, bo):
    idx = x[:, :MAXD].astype(jnp.int32).reshape(NW, NCHUNK, CHUNK)
    num = x[:, MAXD:]

    e = _build_sc_gather()(idx, emb).reshape(BATCH, MAXD * EDIM)

    W1t = W1.T
    w1e = W1t[:MAXD * EDIM]
    w1n = W1t[MAXD * EDIM:]

    out = pl.pallas_call(
        _mlp_body,
        out_shape=jax.ShapeDtypeStruct((BATCH, 2), jnp.float32),
    )(
        e, num, w1e, w1n,
        b1.reshape(1, -1), g1.reshape(1, -1), be1.reshape(1, -1),
        W2.T, b2.reshape(1, -1), g2.reshape(1, -1), be2.reshape(1, -1),
        W3.T, b3.reshape(1, -1), g3.reshape(1, -1), be3.reshape(1, -1),
        Wo.T, bo.reshape(1, -1),
    )
    return out


# speedup vs baseline: 1.5695x; 1.5695x over previous
import functools

import jax
import jax.numpy as jnp
from jax import lax
from jax.experimental import pallas as pl
from jax.experimental.pallas import tpu as pltpu
from jax.experimental.pallas import tpu_sc as plsc

MAXD = 10
EDIM = 64
BATCH = 4096
EPS = 1e-5

NC, NS = 2, 16
NW = NC * NS
NIDX = BATCH * MAXD
CHUNK = 128
NCHUNK = NIDX // (NW * CHUNK)


def _gather_body(idx_hbm, table_hbm, out_hbm, idx_v, rows_v, sem):
    wid = lax.axis_index("s") * NC + lax.axis_index("c")
    pltpu.sync_copy(idx_hbm.at[wid], idx_v)

    def do_chunk(j, carry):
        def fire(g, c):
            vec = idx_v[j, pl.ds(g * 16, 16)]
            base = g * 16
            for k in range(16):
                pltpu.async_copy(table_hbm.at[vec[k]], rows_v.at[base + k], sem)
            return c
        lax.fori_loop(0, CHUNK // 16, fire, 0)
        pltpu.make_async_copy(out_hbm.at[wid, j], rows_v, sem).wait()
        pltpu.sync_copy(rows_v, out_hbm.at[wid, j])
        return carry

    lax.fori_loop(0, NCHUNK, do_chunk, 0)


@functools.cache
def _build_sc_gather():
    return pl.kernel(
        _gather_body,
        out_type=jax.ShapeDtypeStruct((NW, NCHUNK, CHUNK, EDIM), jnp.float32),
        mesh=plsc.VectorSubcoreMesh(
            core_axis_name="c", subcore_axis_name="s",
            num_cores=NC, num_subcores=NS,
        ),
        scratch_types=[
            pltpu.VMEM((NCHUNK, CHUNK), jnp.int32),
            pltpu.VMEM((CHUNK, EDIM), jnp.float32),
            pltpu.SemaphoreType.DMA,
        ],
    )


def _bn_relu(h, g, be):
    m = jnp.mean(h, axis=0, keepdims=True)
    c = h - m
    v = jnp.mean(c * c, axis=0, keepdims=True)
    return jnp.maximum(g * c * jax.lax.rsqrt(v + EPS) + be, 0.0)


def _mlp_body(e_ref, num_ref, w1e_ref, w1n_ref, b1_ref, g1_ref, be1_ref,
              w2_ref, b2_ref, g2_ref, be2_ref,
              w3_ref, b3_ref, g3_ref, be3_ref,
              wo_ref, bo_ref, out_ref):
    f32 = jnp.float32
    h1 = (jnp.dot(e_ref[...], w1e_ref[...], preferred_element_type=f32)
          + jnp.dot(num_ref[...], w1n_ref[...], preferred_element_type=f32)
          + b1_ref[...])
    h1 = _bn_relu(h1, g1_ref[...], be1_ref[...])
    h2 = jnp.dot(h1, w2_ref[...], preferred_element_type=f32) + b2_ref[...]
    h2 = _bn_relu(h2, g2_ref[...], be2_ref[...])
    h3 = jnp.dot(h2, w3_ref[...], preferred_element_type=f32) + b3_ref[...]
    h3 = _bn_relu(h3, g3_ref[...], be3_ref[...])
    out_ref[...] = (jnp.dot(h3, wo_ref[...], preferred_element_type=f32)
                    + bo_ref[...])


def kernel(x, emb, W1, b1, g1, be1, W2, b2, g2, be2, W3, b3, g3, be3, Wo, bo):
    idx = x[:, :MAXD].astype(jnp.int32).reshape(NW, NCHUNK, CHUNK)
    num = x[:, MAXD:]

    e = _build_sc_gather()(idx, emb).reshape(BATCH, MAXD * EDIM)

    W1t = W1.T
    w1e = W1t[:MAXD * EDIM]
    w1n = W1t[MAXD * EDIM:]

    out = pl.pallas_call(
        _mlp_body,
        out_shape=jax.ShapeDtypeStruct((BATCH, 2), jnp.float32),
    )(
        e, num, w1e, w1n,
        b1.reshape(1, -1), g1.reshape(1, -1), be1.reshape(1, -1),
        W2.T, b2.reshape(1, -1), g2.reshape(1, -1), be2.reshape(1, -1),
        W3.T, b3.reshape(1, -1), g3.reshape(1, -1), be3.reshape(1, -1),
        Wo.T, bo.reshape(1, -1),
    )
    return out
